# Initial kernel scaffold; baseline (speedup 1.0000x reference)
import jax
import jax.numpy as jnp
from jax.experimental import pallas as pl


def kernel(x, x_raw, m_edges, t_edges, W1, b1, W2, b2, W3, b3):
    raise NotImplementedError("write your pallas kernel here")


# speedup vs baseline: 4.4581x; 4.4581x over previous
import jax
import jax.numpy as jnp
from jax.experimental import pallas as pl


def _moe_body(x_ref, xr_ref, me_ref, te_ref, w1_ref, b1_ref, w2_ref,
              b2_ref, w3_ref, b3_ref, out_ref):
    e = pl.program_id(0)
    n_t = te_ref.shape[1] + 1

    xr0 = xr_ref[:, 0:1]
    xr1 = xr_ref[:, 1:2]
    m_bins = jnp.sum((xr0 > me_ref[:, :]).astype(jnp.int32), axis=1,
                     keepdims=True)
    t_bins = jnp.sum((xr1 > te_ref[:, :]).astype(jnp.int32), axis=1,
                     keepdims=True)
    flat_idx = m_bins * n_t + t_bins

    h = jnp.dot(x_ref[:, :], w1_ref[0], preferred_element_type=jnp.float32)
    h = jnp.maximum(h + b1_ref[0], 0.0)
    h = jnp.dot(h, w2_ref[0], preferred_element_type=jnp.float32)
    h = jnp.maximum(h + b2_ref[0], 0.0)
    o = jnp.dot(h, w3_ref[0], preferred_element_type=jnp.float32)
    o = o + b3_ref[0]

    contrib = jnp.where(flat_idx == e, o, 0.0)

    @pl.when(e == 0)
    def _():
        out_ref[:, :] = contrib

    @pl.when(e != 0)
    def _():
        out_ref[:, :] = out_ref[:, :] + contrib


def kernel(x, x_raw, m_edges, t_edges, W1, b1, W2, b2, W3, b3):
    B, D = x.shape
    E, _, H = W1.shape

    xr = x_raw[:, :2]
    me = m_edges.reshape(1, -1)
    te = t_edges.reshape(1, -1)
    b1r = b1.reshape(E, 1, H)
    b2r = b2.reshape(E, 1, H)
    b3r = b3.reshape(E, 1, 1)

    out = pl.pallas_call(
        _moe_body,
        grid=(E,),
        in_specs=[
            pl.BlockSpec((B, D), lambda e: (0, 0)),
            pl.BlockSpec((B, 2), lambda e: (0, 0)),
            pl.BlockSpec(me.shape, lambda e: (0, 0)),
            pl.BlockSpec(te.shape, lambda e: (0, 0)),
            pl.BlockSpec((1, D, H), lambda e: (e, 0, 0)),
            pl.BlockSpec((1, 1, H), lambda e: (e, 0, 0)),
            pl.BlockSpec((1, H, H), lambda e: (e, 0, 0)),
            pl.BlockSpec((1, 1, H), lambda e: (e, 0, 0)),
            pl.BlockSpec((1, H, 1), lambda e: (e, 0, 0)),
            pl.BlockSpec((1, 1, 1), lambda e: (e, 0, 0)),
        ],
        out_specs=pl.BlockSpec((B, 1), lambda e: (0, 0)),
        out_shape=jax.ShapeDtypeStruct((B, 1), jnp.float32),
    )(x, xr, me, te, W1, b1r, W2, b2r, W3, b3r)
    return out


# speedup vs baseline: 8.3113x; 1.8643x over previous
import jax
import jax.numpy as jnp
from jax.experimental import pallas as pl
from jax.experimental.pallas import tpu as pltpu


def _moe_body(x_ref, xr_ref, me_ref, te_ref, w1_ref, b1_ref, w2_ref,
              b2_ref, w3_ref, b3_ref, out_ref, flat_ref):
    e = pl.program_id(0)
    n_t = te_ref.shape[1] + 1

    @pl.when(e == 0)
    def _():
        xr0 = xr_ref[:, 0:1]
        xr1 = xr_ref[:, 1:2]
        m_bins = jnp.sum((xr0 > me_ref[:, :]).astype(jnp.int32), axis=1,
                         keepdims=True)
        t_bins = jnp.sum((xr1 > te_ref[:, :]).astype(jnp.int32), axis=1,
                         keepdims=True)
        flat_ref[:, :] = m_bins * n_t + t_bins

    h = jnp.dot(x_ref[:, :], w1_ref[0], preferred_element_type=jnp.float32)
    h = jnp.maximum(h + b1_ref[0], 0.0)
    h = jnp.dot(h, w2_ref[0], preferred_element_type=jnp.float32)
    h = jnp.maximum(h + b2_ref[0], 0.0)
    o = jnp.dot(h, w3_ref[0], preferred_element_type=jnp.float32)
    o = o + b3_ref[0]

    contrib = jnp.where(flat_ref[:, :] == e, o, 0.0)

    @pl.when(e == 0)
    def _():
        out_ref[:, :] = contrib

    @pl.when(e != 0)
    def _():
        out_ref[:, :] = out_ref[:, :] + contrib


def kernel(x, x_raw, m_edges, t_edges, W1, b1, W2, b2, W3, b3):
    B, D = x.shape
    E, _, H = W1.shape

    me = m_edges.reshape(1, -1)
    te = t_edges.reshape(1, -1)
    b1r = b1.reshape(E, 1, H)
    b2r = b2.reshape(E, 1, H)
    b3r = b3.reshape(E, 1, 1)

    out = pl.pallas_call(
        _moe_body,
        grid=(E,),
        in_specs=[
            pl.BlockSpec((B, D), lambda e: (0, 0)),
            pl.BlockSpec((B, D), lambda e: (0, 0)),
            pl.BlockSpec(me.shape, lambda e: (0, 0)),
            pl.BlockSpec(te.shape, lambda e: (0, 0)),
            pl.BlockSpec((1, D, H), lambda e: (e, 0, 0)),
            pl.BlockSpec((1, 1, H), lambda e: (e, 0, 0)),
            pl.BlockSpec((1, H, H), lambda e: (e, 0, 0)),
            pl.BlockSpec((1, 1, H), lambda e: (e, 0, 0)),
            pl.BlockSpec((1, H, 1), lambda e: (e, 0, 0)),
            pl.BlockSpec((1, 1, 1), lambda e: (e, 0, 0)),
        ],
        out_specs=pl.BlockSpec((B, 1), lambda e: (0, 0)),
        out_shape=jax.ShapeDtypeStruct((B, 1), jnp.float32),
        scratch_shapes=[pltpu.VMEM((B, 1), jnp.int32)],
    )(x, x_raw, me, te, W1, b1r, W2, b2r, W3, b3r)
    return out


# speedup vs baseline: 15.3789x; 1.8504x over previous
import jax
import jax.numpy as jnp
from jax.experimental import pallas as pl
from jax.experimental.pallas import tpu as pltpu


def _moe_body(xt_ref, xrt_ref, me_ref, te_ref, w1_ref, b1_ref, w2_ref,
              b2_ref, w3_ref, b3_ref, out_ref, flat_ref):
    e = pl.program_id(0)
    n_m = me_ref.shape[0] + 1
    n_t = te_ref.shape[0] + 1

    @pl.when(e == 0)
    def _():
        xr0 = xrt_ref[0:1, :]
        xr1 = xrt_ref[1:2, :]
        m_bins = jnp.zeros_like(xr0, dtype=jnp.int32)
        for j in range(n_m - 1):
            m_bins = m_bins + (xr0 > me_ref[j]).astype(jnp.int32)
        t_bins = jnp.zeros_like(xr1, dtype=jnp.int32)
        for j in range(n_t - 1):
            t_bins = t_bins + (xr1 > te_ref[j]).astype(jnp.int32)
        flat_ref[:, :] = m_bins * n_t + t_bins

    dn = (((0,), (0,)), ((), ()))
    h = jax.lax.dot_general(w1_ref[0], xt_ref[:, :], dn,
                            preferred_element_type=jnp.float32)
    h = jnp.maximum(h + b1_ref[0], 0.0)
    h = jax.lax.dot_general(w2_ref[0], h, dn,
                            preferred_element_type=jnp.float32)
    h = jnp.maximum(h + b2_ref[0], 0.0)
    o = jax.lax.dot_general(w3_ref[0], h, dn,
                            preferred_element_type=jnp.float32)
    o = o + b3_ref[0]

    contrib = jnp.where(flat_ref[:, :] == e, o, 0.0)

    @pl.when(e == 0)
    def _():
        out_ref[:, :] = contrib

    @pl.when(e != 0)
    def _():
        out_ref[:, :] = out_ref[:, :] + contrib


def kernel(x, x_raw, m_edges, t_edges, W1, b1, W2, b2, W3, b3):
    B, D = x.shape
    E, _, H = W1.shape

    xt = x.T
    xrt = x_raw[:, :2].T
    b1r = b1.reshape(E, H, 1)
    b2r = b2.reshape(E, H, 1)
    b3r = b3.reshape(E, 1, 1)

    out = pl.pallas_call(
        _moe_body,
        grid=(E,),
        in_specs=[
            pl.BlockSpec((D, B), lambda e: (0, 0)),
            pl.BlockSpec((2, B), lambda e: (0, 0)),
            pl.BlockSpec(memory_space=pltpu.SMEM),
            pl.BlockSpec(memory_space=pltpu.SMEM),
            pl.BlockSpec((1, D, H), lambda e: (e, 0, 0)),
            pl.BlockSpec((1, H, 1), lambda e: (e, 0, 0)),
            pl.BlockSpec((1, H, H), lambda e: (e, 0, 0)),
            pl.BlockSpec((1, H, 1), lambda e: (e, 0, 0)),
            pl.BlockSpec((1, H, 1), lambda e: (e, 0, 0)),
            pl.BlockSpec((1, 1, 1), lambda e: (e, 0, 0)),
        ],
        out_specs=pl.BlockSpec((1, B), lambda e: (0, 0)),
        out_shape=jax.ShapeDtypeStruct((1, B), jnp.float32),
        scratch_shapes=[pltpu.VMEM((1, B), jnp.int32)],
    )(xt, xrt, m_edges, t_edges, W1, b1r, W2, b2r, W3, b3r)
    return out.reshape(B, 1)
